# baseline (speedup 1.0000x reference)
import jax
import jax.numpy as jnp
from jax.experimental import pallas as pl
from jax.experimental.pallas import tpu as pltpu

_NEG_SLOPE = 0.2


def _lrelu(v):
    return jnp.maximum(v, _NEG_SLOPE * v)


def _round_up(n, m):
    return ((n + m - 1) // m) * m


def _dot0(a, b):
    return jax.lax.dot_general(a, b, (((0,), (0,)), ((), ())),
                               preferred_element_type=jnp.float32)


def _fused_kernel(pos_ref, x_ref, w11_ref, b11_ref, w2a1_ref, b2a1_ref,
                  w2b1_ref, ws1_ref, bs1_ref, b2b1_ref, ws2_ref, bs2_ref,
                  w12_ref, b12_ref, w2a2_ref, b2a2_ref, w2b2_ref, b2b2_ref,
                  w3_ref, b3_ref, o_ref, s_ref):
    f32 = jnp.float32
    half = pos_ref.shape[2]
    T = 2 * half
    z = lambda r, c: jnp.zeros((r, c), f32)
    row = lambda v: v[None, :]

    p = jnp.concatenate([pos_ref[0], pos_ref[1]], axis=1)
    lane = jax.lax.broadcasted_iota(jnp.int32, (1, T), 1)
    ones_a = (lane < half).astype(f32)
    s_ref[0:8, :] = jnp.concatenate(
        [p, ones_a, 1.0 - ones_a, z(3, T)], axis=0)
    s_ref[8:136, 0:half] = x_ref[0]
    s_ref[8:136, half:T] = x_ref[1]

    b11r = row(b11_ref[...])
    w_h1 = jnp.concatenate([w11_ref[...], b11r, b11r, z(3, 64)],
                           axis=0)
    h1 = _lrelu(_dot0(w_h1, s_ref[0:8, :]))
    h2 = _lrelu(_dot0(w2a1_ref[...], h1) + b2a1_ref[...][:, None])
    bsb = row(bs1_ref[...] + b2b1_ref[...])
    w_s1 = jnp.concatenate([ws1_ref[...], bsb, bsb, z(3, 128)],
                           axis=0)
    s1b = _dot0(w_s1, s_ref[0:8, :])
    y1 = _dot0(w2b1_ref[...], h2) + s1b

    pooled = jnp.concatenate(
        [jnp.max(y1[:, 0:half], axis=1, keepdims=True),
         jnp.max(y1[:, half:T], axis=1, keepdims=True)], axis=1)
    pooled_aug = jnp.concatenate([pooled, jnp.ones((1, 2), f32)],
                                 axis=0)
    w2 = jnp.concatenate([ws2_ref[...], w12_ref[...]], axis=1)
    bc2r = row(jnp.concatenate([bs2_ref[...], b12_ref[...]]))
    w2_feat_aug = jnp.concatenate([w2[3:131, :], bc2r], axis=0)
    pt_rows = _dot0(pooled_aug, w2_feat_aug)

    w_xc2 = jnp.concatenate(
        [w2[0:3, :], pt_rows, z(3, 192), w2[131:259, :]],
        axis=0)
    xc2 = _dot0(w_xc2, s_ref[0:136, :])
    h = _lrelu(xc2[64:192, :])
    g2 = _lrelu(_dot0(w2a2_ref[...], h) + b2a2_ref[...][:, None])
    s2b = xc2[0:64, :] + b2b2_ref[...][:, None]
    y2 = _dot0(w2b2_ref[...], g2) + s2b
    feat = _lrelu(y2)
    out = _dot0(w3_ref[...], feat) + b3_ref[...]
    o_ref[0] = out[:, 0:half].astype(o_ref.dtype)
    o_ref[1] = out[:, half:T].astype(o_ref.dtype)


def kernel(pos, x, ws1, bs1, w11, b11, w2a1, b2a1, w2b1, b2b1, ws2, bs2,
           w12, b12, w2a2, b2a2, w2b2, b2b2, w3, b3):
    f32 = jnp.float32
    B, cp, N = pos.shape
    cx = x.shape[1]
    c3_out = w3.shape[1]

    n_pad = _round_up(N, 128)
    pad = n_pad - N
    if pad:
        pos = jnp.pad(pos, ((0, 0), (0, 0), (0, pad)), mode="edge")
        x = jnp.pad(x, ((0, 0), (0, 0), (0, pad)), mode="edge")
    if B % 2:
        pos = jnp.concatenate([pos, pos[-1:]], axis=0)
        x = jnp.concatenate([x, x[-1:]], axis=0)
    bp = pos.shape[0]

    flops = 2 * B * n_pad * (8 * 64 + 72 * 128 + 136 * 128
                             + 136 * 192 + 136 * 64 + 200 * 64 + 64 * 3)
    bytes_accessed = int(4 * (pos.size + x.size + B * c3_out * n_pad))


    full = lambda *s: pl.BlockSpec(s, lambda i: (0,) * len(s))
    out_pad = pl.pallas_call(
        _fused_kernel,
        out_shape=jax.ShapeDtypeStruct((bp, c3_out, n_pad), pos.dtype),
        grid=(bp // 2,),
        in_specs=[
            pl.BlockSpec((2, cp, n_pad), lambda i: (i, 0, 0)),
            pl.BlockSpec((2, cx, n_pad), lambda i: (i, 0, 0)),
            full(cp, 64), full(64,),
            full(64, 128), full(128,),
            full(128, 128), full(cp, 128), full(128,), full(128,),
            full(cp + 128 + cx, 64), full(64,),
            full(cp + 128 + cx, 128), full(128,),
            full(128, 64), full(64,),
            full(64, 64), full(64,),
            full(64, c3_out), full(c3_out, 1),
        ],
        out_specs=pl.BlockSpec((2, c3_out, n_pad), lambda i: (i, 0, 0)),
        scratch_shapes=[pltpu.VMEM((136, 2 * n_pad), f32)],
        compiler_params=pltpu.CompilerParams(
            dimension_semantics=("arbitrary",),
            vmem_limit_bytes=56 * 1024 * 1024),
        cost_estimate=pl.CostEstimate(flops=flops, transcendentals=0,
                                      bytes_accessed=bytes_accessed),
    )(pos.astype(f32), x.astype(f32), w11.astype(f32), b11.astype(f32),
      w2a1.astype(f32), b2a1.astype(f32), w2b1.astype(f32), ws1.astype(f32),
      bs1.astype(f32), b2b1.astype(f32), ws2.astype(f32), bs2.astype(f32),
      w12.astype(f32), b12.astype(f32), w2a2.astype(f32), b2a2.astype(f32),
      w2b2.astype(f32), b2b2.astype(f32), w3.astype(f32),
      b3.astype(f32)[:, None])

    out_pad = out_pad[:B]
    return out_pad[:, :, :N] if pad else out_pad


# speedup vs baseline: 1.0816x; 1.0816x over previous
import jax
import jax.numpy as jnp
from jax.experimental import pallas as pl
from jax.experimental.pallas import tpu as pltpu

_NEG_SLOPE = 0.2


def _lrelu(v):
    return jnp.maximum(v, _NEG_SLOPE * v)


def _round_up(n, m):
    return ((n + m - 1) // m) * m


def _dot0(a, b):
    return jax.lax.dot_general(a, b, (((0,), (0,)), ((), ())),
                               preferred_element_type=jnp.float32)


def _fused_kernel(pos_ref, x_ref, w11_ref, b11_ref, w2a1_ref, b2a1_ref,
                  w2b1_ref, ws1_ref, bs1_ref, b2b1_ref, ws2_ref, bs2_ref,
                  w12_ref, b12_ref, w2a2_ref, b2a2_ref, w2b2_ref, b2b2_ref,
                  w3_ref, b3_ref, o_ref, s_ref):
    f32 = jnp.float32
    half = pos_ref.shape[2]
    T = 2 * half
    z = lambda r, c: jnp.zeros((r, c), f32)
    row = lambda v: v[None, :]

    p = jnp.concatenate([pos_ref[0], pos_ref[1]], axis=1)
    lane = jax.lax.broadcasted_iota(jnp.int32, (1, T), 1)
    ones_a = (lane < half).astype(f32)
    s_ref[128:136, :] = jnp.concatenate(
        [p, ones_a, 1.0 - ones_a, z(3, T)], axis=0)
    s_ref[136:264, 0:half] = x_ref[0]
    s_ref[136:264, half:T] = x_ref[1]

    b11r = row(b11_ref[...])
    w_h1 = jnp.concatenate([w11_ref[...], b11r, b11r, z(3, 64)],
                           axis=0)
    h1 = _lrelu(_dot0(w_h1, s_ref[128:136, :]))
    h2 = _lrelu(_dot0(w2a1_ref[...], h1) + b2a1_ref[...][:, None])
    s_ref[0:128, :] = h2
    bsb = row(bs1_ref[...] + b2b1_ref[...])
    w_y1 = jnp.concatenate(
        [w2b1_ref[...], ws1_ref[...], bsb, bsb, z(3, 128)],
        axis=0)
    y1 = _dot0(w_y1, s_ref[0:136, :])

    pooled = jnp.concatenate(
        [jnp.max(y1[:, 0:half], axis=1, keepdims=True),
         jnp.max(y1[:, half:T], axis=1, keepdims=True)], axis=1)
    pooled_aug = jnp.concatenate([pooled, jnp.ones((1, 2), f32)],
                                 axis=0)
    w2 = jnp.concatenate([ws2_ref[...], w12_ref[...]], axis=1)
    bc2r = row(jnp.concatenate([bs2_ref[...], b12_ref[...]]))
    w2_feat_aug = jnp.concatenate([w2[3:131, :], bc2r], axis=0)
    pt_rows = _dot0(pooled_aug, w2_feat_aug)

    w_xc2 = jnp.concatenate(
        [w2[0:3, :], pt_rows, z(3, 192), w2[131:259, :]],
        axis=0)
    xc2 = _dot0(w_xc2, s_ref[128:264, :])
    h = _lrelu(xc2[64:192, :])
    g2 = _lrelu(_dot0(w2a2_ref[...], h) + b2a2_ref[...][:, None])
    s_ref[264:328, :] = g2
    w_y2 = jnp.concatenate(
        [w2[0:3, 0:64], pt_rows[:, 0:64] + row(b2b2_ref[...]), z(3, 64),
         w2[131:259, 0:64], w2b2_ref[...]], axis=0)
    y2 = _dot0(w_y2, s_ref[128:328, :])
    feat = _lrelu(y2)
    out = _dot0(w3_ref[...], feat) + b3_ref[...]
    o_ref[0] = out[:, 0:half].astype(o_ref.dtype)
    o_ref[1] = out[:, half:T].astype(o_ref.dtype)


def kernel(pos, x, ws1, bs1, w11, b11, w2a1, b2a1, w2b1, b2b1, ws2, bs2,
           w12, b12, w2a2, b2a2, w2b2, b2b2, w3, b3):
    f32 = jnp.float32
    B, cp, N = pos.shape
    cx = x.shape[1]
    c3_out = w3.shape[1]

    n_pad = _round_up(N, 128)
    pad = n_pad - N
    if pad:
        pos = jnp.pad(pos, ((0, 0), (0, 0), (0, pad)), mode="edge")
        x = jnp.pad(x, ((0, 0), (0, 0), (0, pad)), mode="edge")
    if B % 2:
        pos = jnp.concatenate([pos, pos[-1:]], axis=0)
        x = jnp.concatenate([x, x[-1:]], axis=0)
    bp = pos.shape[0]

    flops = 2 * B * n_pad * (8 * 64 + 72 * 128 + 136 * 128
                             + 136 * 192 + 136 * 64 + 200 * 64 + 64 * 3)
    bytes_accessed = int(4 * (pos.size + x.size + B * c3_out * n_pad))


    full = lambda *s: pl.BlockSpec(s, lambda i: (0,) * len(s))
    out_pad = pl.pallas_call(
        _fused_kernel,
        out_shape=jax.ShapeDtypeStruct((bp, c3_out, n_pad), pos.dtype),
        grid=(bp // 2,),
        in_specs=[
            pl.BlockSpec((2, cp, n_pad), lambda i: (i, 0, 0)),
            pl.BlockSpec((2, cx, n_pad), lambda i: (i, 0, 0)),
            full(cp, 64), full(64,),
            full(64, 128), full(128,),
            full(128, 128), full(cp, 128), full(128,), full(128,),
            full(cp + 128 + cx, 64), full(64,),
            full(cp + 128 + cx, 128), full(128,),
            full(128, 64), full(64,),
            full(64, 64), full(64,),
            full(64, c3_out), full(c3_out, 1),
        ],
        out_specs=pl.BlockSpec((2, c3_out, n_pad), lambda i: (i, 0, 0)),
        scratch_shapes=[pltpu.VMEM((328, 2 * n_pad), f32)],
        compiler_params=pltpu.CompilerParams(
            dimension_semantics=("arbitrary",),
            vmem_limit_bytes=56 * 1024 * 1024),
        cost_estimate=pl.CostEstimate(flops=flops, transcendentals=0,
                                      bytes_accessed=bytes_accessed),
    )(pos.astype(f32), x.astype(f32), w11.astype(f32), b11.astype(f32),
      w2a1.astype(f32), b2a1.astype(f32), w2b1.astype(f32), ws1.astype(f32),
      bs1.astype(f32), b2b1.astype(f32), ws2.astype(f32), bs2.astype(f32),
      w12.astype(f32), b12.astype(f32), w2a2.astype(f32), b2a2.astype(f32),
      w2b2.astype(f32), b2b2.astype(f32), w3.astype(f32),
      b3.astype(f32)[:, None])

    out_pad = out_pad[:B]
    return out_pad[:, :, :N] if pad else out_pad
